# baseline (speedup 1.0000x reference)
import jax
import jax.numpy as jnp
from jax import lax
from jax.experimental import pallas as pl
from jax.experimental.pallas import tpu as pltpu
from jax.experimental.pallas import tpu_sc as plsc

N = 50000
HID = 64
HALF = 32
E = 800000
NS = 16
CHUNK = 128
SLAB = 28
NSLAB = 14
EPT = NSLAB * SLAB * CHUNK
E_PAD = NS * EPT
NBLK = NS * NSLAB
ACC_ROWS = 51200
ZCH = ACC_ROWS // NS
SINK = N
ROWBLK = 1000
GRID = N // ROWBLK


def _agg_pass(s, tab_ref, src_ref, dst_ref, out_ref, acc, srcs, dsts,
              rows, sems, zeros_ref):
    base = s * ZCH
    pltpu.sync_copy(zeros_ref, rows[0])

    @pl.loop(0, ZCH // CHUNK)
    def _z(k):
        pltpu.sync_copy(rows[0], acc.at[pl.ds(base + k * CHUNK, CHUNK)])

    plsc.subcore_barrier()

    def _gwait(sem):
        pltpu.make_async_copy(
            tab_ref.at[pl.ds(0, CHUNK)], rows[0], sem).wait()

    @pl.loop(0, NSLAB)
    def _slab(t):
        blk = s * NSLAB + t
        pltpu.sync_copy(src_ref.at[blk], srcs)
        pltpu.sync_copy(dst_ref.at[blk], dsts)
        for u in range(3):
            pltpu.async_copy(tab_ref.at[srcs.at[u]], rows[u], sems[u])

        @pl.loop(0, SLAB // 4)
        def _quad(i):
            j0 = 4 * i
            for u in range(4):
                _gwait(sems[u])
                nxt = j0 + u + 3
                if u == 0:
                    pltpu.async_copy(
                        tab_ref.at[srcs.at[nxt]], rows[3], sems[3])
                else:
                    @pl.when(i < SLAB // 4 - 1)
                    def _():
                        pltpu.async_copy(
                            tab_ref.at[srcs.at[nxt]], rows[u - 1],
                            sems[u - 1])

                pltpu.sync_copy(rows[u], acc.at[dsts.at[j0 + u]], add=True)

    plsc.subcore_barrier()
    @pl.loop(0, ZCH // CHUNK)
    def _wb(k):
        r = base + k * CHUNK
        pltpu.sync_copy(acc.at[pl.ds(r, CHUNK)], rows[0])
        pltpu.sync_copy(rows[0], out_ref.at[pl.ds(r, CHUNK)])

    plsc.subcore_barrier()


def _sc_agg_body(xf4, sct, dct, stc, dtc, zeros_ref, sums4,
                 acc, srcs, dsts, r0, r1, r2, r3, m0, m1, m2, m3):
    c = lax.axis_index("c")
    s = lax.axis_index("s")
    rows = [r0, r1, r2, r3]
    sems = [m0, m1, m2, m3]

    @pl.when(c == 0)
    def _():
        _agg_pass(s, xf4.at[0], sct, dct, sums4.at[0], acc, srcs, dsts,
                  rows, sems, zeros_ref)
        _agg_pass(s, xf4.at[2], stc, dtc, sums4.at[2], acc, srcs, dsts,
                  rows, sems, zeros_ref)

    @pl.when(c == 1)
    def _():
        _agg_pass(s, xf4.at[1], sct, dct, sums4.at[1], acc, srcs, dsts,
                  rows, sems, zeros_ref)
        _agg_pass(s, xf4.at[3], stc, dtc, sums4.at[3], acc, srcs, dsts,
                  rows, sems, zeros_ref)


def _make_sc_agg():
    mesh = plsc.VectorSubcoreMesh(core_axis_name="c", subcore_axis_name="s")
    return pl.kernel(
        _sc_agg_body,
        out_type=jax.ShapeDtypeStruct((4, ACC_ROWS, HALF), jnp.float32),
        mesh=mesh,
        compiler_params=pltpu.CompilerParams(use_tc_tiling_on_sc=False),
        scratch_types=[
            pltpu.VMEM_SHARED((ACC_ROWS, HALF), jnp.float32),
            pltpu.VMEM((SLAB, CHUNK), jnp.int32),
            pltpu.VMEM((SLAB, CHUNK), jnp.int32),
            pltpu.VMEM((CHUNK, HALF), jnp.float32),
            pltpu.VMEM((CHUNK, HALF), jnp.float32),
            pltpu.VMEM((CHUNK, HALF), jnp.float32),
            pltpu.VMEM((CHUNK, HALF), jnp.float32),
            pltpu.SemaphoreType.DMA,
            pltpu.SemaphoreType.DMA,
            pltpu.SemaphoreType.DMA,
            pltpu.SemaphoreType.DMA,
        ],
    )


def _cnt_pass(s, dst_ref, out_ref, acc, dsts, ones_v, zeros_ref, ones_ref):
    base = s * ZCH
    pltpu.sync_copy(zeros_ref, ones_v)

    @pl.loop(0, ZCH // CHUNK)
    def _z(k):
        pltpu.sync_copy(ones_v, acc.at[pl.ds(base + k * CHUNK, CHUNK)])

    pltpu.sync_copy(ones_ref, ones_v)
    plsc.subcore_barrier()

    @pl.loop(0, NSLAB)
    def _slab(t):
        blk = s * NSLAB + t
        pltpu.sync_copy(dst_ref.at[blk], dsts)

        @pl.loop(0, SLAB)
        def _chunk(j):
            pltpu.sync_copy(ones_v, acc.at[dsts.at[j]], add=True)

    plsc.subcore_barrier()

    @pl.loop(0, ZCH // CHUNK)
    def _wb(k):
        r = base + k * CHUNK
        pltpu.sync_copy(acc.at[pl.ds(r, CHUNK)], ones_v)
        pltpu.sync_copy(ones_v, out_ref.at[pl.ds(r, CHUNK)])


def _sc_cnt_body(dct, dtc, ones_ref, zeros_ref, cnt2, acc, dsts, ones_v):
    c = lax.axis_index("c")
    s = lax.axis_index("s")

    @pl.when(c == 0)
    def _():
        _cnt_pass(s, dct, cnt2.at[0], acc, dsts, ones_v, zeros_ref, ones_ref)

    @pl.when(c == 1)
    def _():
        _cnt_pass(s, dtc, cnt2.at[1], acc, dsts, ones_v, zeros_ref, ones_ref)


def _make_sc_cnt():
    mesh = plsc.VectorSubcoreMesh(core_axis_name="c", subcore_axis_name="s")
    return pl.kernel(
        _sc_cnt_body,
        out_type=jax.ShapeDtypeStruct((2, ACC_ROWS, 16), jnp.float32),
        mesh=mesh,
        compiler_params=pltpu.CompilerParams(use_tc_tiling_on_sc=False),
        scratch_types=[
            pltpu.VMEM_SHARED((ACC_ROWS, 16), jnp.float32),
            pltpu.VMEM((SLAB, CHUNK), jnp.int32),
            pltpu.VMEM((CHUNK, 16), jnp.float32),
        ],
    )


def _dot(a, b):
    return jnp.dot(a, b, preferred_element_type=jnp.float32,
                   precision=jax.lax.Precision.HIGHEST)


def _enc_body(x_ref, w_ref, b_ref, o_ref):
    h = jnp.maximum(_dot(x_ref[0], w_ref[0]) + b_ref[0], 0.0)
    o_ref[0] = h[:, :HALF]
    o_ref[1] = h[:, HALF:]


def _encode(x, w, b):
    return pl.pallas_call(
        _enc_body,
        grid=(2, GRID),
        in_specs=[
            pl.BlockSpec((1, ROWBLK, 128), lambda t, i: (t, i, 0)),
            pl.BlockSpec((1, 128, HID), lambda t, i: (t, 0, 0)),
            pl.BlockSpec((1, 1, HID), lambda t, i: (t, 0, 0)),
        ],
        out_specs=pl.BlockSpec((2, ROWBLK, HALF), lambda t, i: (t, i, 0)),
        out_shape=jax.ShapeDtypeStruct((4, N, HALF), jnp.float32),
    )(x, w, b.reshape(2, 1, HID))


def _dense_body(s_ref, cnt_ref, x_ref, wl_ref, bl_ref, wr_ref,
                w1_ref, b1_ref, w2_ref, b2_ref, y_ref, o_ref):
    cv = jnp.maximum(cnt_ref[0][:, :1], 1.0)
    m = jnp.concatenate([s_ref[0], s_ref[1]], axis=1) / cv
    x = jnp.concatenate([x_ref[0], x_ref[1]], axis=1)
    h = jnp.maximum(_dot(m, wl_ref[0]) + bl_ref[0] + _dot(x, wr_ref[0]),
                    0.0) + x
    y_ref[0] = h[:, :HALF]
    y_ref[1] = h[:, HALF:]
    hh = jnp.maximum(_dot(h, w1_ref[...]) + b1_ref[...], 0.0)
    o_ref[...] = _dot(hh, w2_ref[...]) + b2_ref[...]


def _dense(sums4, cnt2, xf4, wl, bl, wr, w1, b1, w2, b2):
    return pl.pallas_call(
        _dense_body,
        grid=(2, GRID),
        in_specs=[
            pl.BlockSpec((2, ROWBLK, HALF), lambda g, i: (g, i, 0)),
            pl.BlockSpec((1, ROWBLK, 16), lambda g, i: (g, i, 0)),
            pl.BlockSpec((2, ROWBLK, HALF), lambda g, i: (1 - g, i, 0)),
            pl.BlockSpec((1, HID, HID), lambda g, i: (g, 0, 0)),
            pl.BlockSpec((1, 1, HID), lambda g, i: (g, 0, 0)),
            pl.BlockSpec((1, HID, HID), lambda g, i: (g, 0, 0)),
            pl.BlockSpec((HID, HID // 2), lambda g, i: (0, 0)),
            pl.BlockSpec((1, HID // 2), lambda g, i: (0, 0)),
            pl.BlockSpec((HID // 2, 1), lambda g, i: (0, 0)),
            pl.BlockSpec((1, 1), lambda g, i: (0, 0)),
        ],
        out_specs=[
            pl.BlockSpec((2, ROWBLK, HALF), lambda g, i: (1 - g, i, 0)),
            pl.BlockSpec((ROWBLK, 1), lambda g, i: (i, 0)),
        ],
        out_shape=[
            jax.ShapeDtypeStruct((4, N, HALF), jnp.float32),
            jax.ShapeDtypeStruct((N, 1), jnp.float32),
        ],
    )(sums4, cnt2, xf4, wl, bl.reshape(2, 1, HID), wr,
      w1, b1.reshape(1, HID // 2), w2, b2.reshape(1, 1))


def _pad_edges(e):
    pad = jnp.broadcast_to(
        jnp.array([[0], [SINK]], dtype=jnp.int32), (2, E_PAD - E))
    ep = jnp.concatenate([e, pad], axis=1)
    return (ep[0].reshape(NBLK, SLAB, CHUNK), ep[1].reshape(NBLK, SLAB, CHUNK))


@jax.jit
def kernel(x_compound, x_target, edge_ct, edge_tc, W_comp, b_comp, W_tgt,
           b_tgt, Wl_ct, bl_ct, Wr_ct, Wl_tc, bl_tc, Wr_tc, W_out1, b_out1,
           W_out2, b_out2):
    sct, dct = _pad_edges(edge_ct)
    stc, dtc = _pad_edges(edge_tc)
    zeros32 = jnp.zeros((CHUNK, HALF), jnp.float32)
    zeros16 = jnp.zeros((CHUNK, 16), jnp.float32)
    ones16 = jnp.ones((CHUNK, 16), jnp.float32)

    X = jnp.stack([x_compound, x_target])
    WE = jnp.stack([W_comp, W_tgt])
    BE = jnp.stack([b_comp, b_tgt])
    xf4 = _encode(X, WE, BE)

    cnt2 = _make_sc_cnt()(dct, dtc, ones16, zeros16)

    sc_agg = _make_sc_agg()
    out = None
    for i in range(Wl_ct.shape[0]):
        sums4 = sc_agg(xf4, sct, dct, stc, dtc, zeros32)
        wl = jnp.stack([Wl_ct[i], Wl_tc[i]])
        bl = jnp.stack([bl_ct[i], bl_tc[i]])
        wr = jnp.stack([Wr_ct[i], Wr_tc[i]])
        xf4, out = _dense(sums4, cnt2, xf4, wl, bl, wr,
                          W_out1, b_out1, W_out2, b_out2)

    return out


# speedup vs baseline: 1.9801x; 1.9801x over previous
import functools

import jax
import jax.numpy as jnp
from jax import lax
from jax.experimental import pallas as pl
from jax.experimental.pallas import tpu as pltpu
from jax.experimental.pallas import tpu_sc as plsc

N = 50000
HID = 64
HALF = 32
E = 800000
NS = 16
CHUNK = 128
SLAB = 28
NSLAB = 14
EPT = NSLAB * SLAB * CHUNK
E_PAD = NS * EPT
NBLK = NS * NSLAB
ACC_ROWS = 51200
ZCH = ACC_ROWS // NS
SINK = N
ROWBLK = 1000
GRID = N // ROWBLK


def _agg_pass(s, tab_ref, src_ref, dst_ref, out_ref, acc, srcs, dsts,
              rows, sems, zeros_ref):
    base = s * ZCH
    pltpu.sync_copy(zeros_ref, rows[0])

    @pl.loop(0, ZCH // CHUNK)
    def _z(k):
        pltpu.sync_copy(rows[0], acc.at[pl.ds(base + k * CHUNK, CHUNK)])

    plsc.subcore_barrier()

    def _gwait(sem):
        pltpu.make_async_copy(
            tab_ref.at[pl.ds(0, CHUNK)], rows[0], sem).wait()

    @pl.loop(0, NSLAB)
    def _slab(t):
        blk = s * NSLAB + t
        pltpu.sync_copy(src_ref.at[blk], srcs)
        pltpu.sync_copy(dst_ref.at[blk], dsts)
        for u in range(3):
            pltpu.async_copy(tab_ref.at[srcs.at[u]], rows[u], sems[u])

        @pl.loop(0, SLAB // 4)
        def _quad(i):
            j0 = 4 * i
            for u in range(4):
                _gwait(sems[u])
                nxt = j0 + u + 3
                if u == 0:
                    pltpu.async_copy(
                        tab_ref.at[srcs.at[nxt]], rows[3], sems[3])
                else:
                    @pl.when(i < SLAB // 4 - 1)
                    def _():
                        pltpu.async_copy(
                            tab_ref.at[srcs.at[nxt]], rows[u - 1],
                            sems[u - 1])

                pltpu.sync_copy(rows[u], acc.at[dsts.at[j0 + u]], add=True)

    plsc.subcore_barrier()
    @pl.loop(0, ZCH // CHUNK)
    def _wb(k):
        r = base + k * CHUNK
        pltpu.sync_copy(acc.at[pl.ds(r, CHUNK)], rows[0])
        pltpu.sync_copy(rows[0], out_ref.at[pl.ds(r, CHUNK)])

    plsc.subcore_barrier()


def _sc_agg_body(xc0, xc1, xt0, xt1, sct, dct, stc, dtc, zeros_ref,
                 ot0, ot1, oc0, oc1, acc, srcs, dsts,
                 r0, r1, r2, r3, m0, m1, m2, m3):
    c = lax.axis_index("c")
    s = lax.axis_index("s")
    rows = [r0, r1, r2, r3]
    sems = [m0, m1, m2, m3]

    @pl.when(c == 0)
    def _():
        _agg_pass(s, xc0, sct, dct, ot0, acc, srcs, dsts, rows, sems,
                  zeros_ref)
        _agg_pass(s, xt0, stc, dtc, oc0, acc, srcs, dsts, rows, sems,
                  zeros_ref)

    @pl.when(c == 1)
    def _():
        _agg_pass(s, xc1, sct, dct, ot1, acc, srcs, dsts, rows, sems,
                  zeros_ref)
        _agg_pass(s, xt1, stc, dtc, oc1, acc, srcs, dsts, rows, sems,
                  zeros_ref)


def _make_sc_agg():
    mesh = plsc.VectorSubcoreMesh(core_axis_name="c", subcore_axis_name="s")
    return pl.kernel(
        _sc_agg_body,
        out_type=[jax.ShapeDtypeStruct((ACC_ROWS, HALF), jnp.float32)] * 4,
        mesh=mesh,
        compiler_params=pltpu.CompilerParams(use_tc_tiling_on_sc=False),
        scratch_types=[
            pltpu.VMEM_SHARED((ACC_ROWS, HALF), jnp.float32),
            pltpu.VMEM((SLAB, CHUNK), jnp.int32),
            pltpu.VMEM((SLAB, CHUNK), jnp.int32),
            pltpu.VMEM((CHUNK, HALF), jnp.float32),
            pltpu.VMEM((CHUNK, HALF), jnp.float32),
            pltpu.VMEM((CHUNK, HALF), jnp.float32),
            pltpu.VMEM((CHUNK, HALF), jnp.float32),
            pltpu.SemaphoreType.DMA,
            pltpu.SemaphoreType.DMA,
            pltpu.SemaphoreType.DMA,
            pltpu.SemaphoreType.DMA,
        ],
    )


def _sc_agg1_body(xt0, xt1, stc, dtc, zeros_ref, oc0, oc1,
                  acc, srcs, dsts, r0, r1, r2, r3, m0, m1, m2, m3):
    c = lax.axis_index("c")
    s = lax.axis_index("s")
    rows = [r0, r1, r2, r3]
    sems = [m0, m1, m2, m3]

    @pl.when(c == 0)
    def _():
        _agg_pass(s, xt0, stc, dtc, oc0, acc, srcs, dsts, rows, sems,
                  zeros_ref)

    @pl.when(c == 1)
    def _():
        _agg_pass(s, xt1, stc, dtc, oc1, acc, srcs, dsts, rows, sems,
                  zeros_ref)


def _make_sc_agg1():
    mesh = plsc.VectorSubcoreMesh(core_axis_name="c", subcore_axis_name="s")
    return pl.kernel(
        _sc_agg1_body,
        out_type=[jax.ShapeDtypeStruct((ACC_ROWS, HALF), jnp.float32)] * 2,
        mesh=mesh,
        compiler_params=pltpu.CompilerParams(use_tc_tiling_on_sc=False),
        scratch_types=[
            pltpu.VMEM_SHARED((ACC_ROWS, HALF), jnp.float32),
            pltpu.VMEM((SLAB, CHUNK), jnp.int32),
            pltpu.VMEM((SLAB, CHUNK), jnp.int32),
            pltpu.VMEM((CHUNK, HALF), jnp.float32),
            pltpu.VMEM((CHUNK, HALF), jnp.float32),
            pltpu.VMEM((CHUNK, HALF), jnp.float32),
            pltpu.VMEM((CHUNK, HALF), jnp.float32),
            pltpu.SemaphoreType.DMA,
            pltpu.SemaphoreType.DMA,
            pltpu.SemaphoreType.DMA,
            pltpu.SemaphoreType.DMA,
        ],
    )


def _cnt_pass(s, dst_ref, out_ref, acc, dsts, ones_v, zeros_ref, ones_ref):
    base = s * ZCH
    pltpu.sync_copy(zeros_ref, ones_v)

    @pl.loop(0, ZCH // CHUNK)
    def _z(k):
        pltpu.sync_copy(ones_v, acc.at[pl.ds(base + k * CHUNK, CHUNK)])

    pltpu.sync_copy(ones_ref, ones_v)
    plsc.subcore_barrier()

    @pl.loop(0, NSLAB)
    def _slab(t):
        blk = s * NSLAB + t
        pltpu.sync_copy(dst_ref.at[blk], dsts)

        @pl.loop(0, SLAB)
        def _chunk(j):
            pltpu.sync_copy(ones_v, acc.at[dsts.at[j]], add=True)

    plsc.subcore_barrier()

    @pl.loop(0, ZCH // CHUNK)
    def _wb(k):
        r = base + k * CHUNK
        pltpu.sync_copy(acc.at[pl.ds(r, CHUNK)], ones_v)
        pltpu.sync_copy(ones_v, out_ref.at[pl.ds(r, CHUNK)])


def _sc_cnt_body(dct, dtc, ones_ref, zeros_ref, cnt_t, cnt_c,
                 acc, dsts, ones_v):
    c = lax.axis_index("c")
    s = lax.axis_index("s")

    @pl.when(c == 0)
    def _():
        _cnt_pass(s, dct, cnt_t, acc, dsts, ones_v, zeros_ref, ones_ref)

    @pl.when(c == 1)
    def _():
        _cnt_pass(s, dtc, cnt_c, acc, dsts, ones_v, zeros_ref, ones_ref)


def _make_sc_cnt():
    mesh = plsc.VectorSubcoreMesh(core_axis_name="c", subcore_axis_name="s")
    return pl.kernel(
        _sc_cnt_body,
        out_type=[jax.ShapeDtypeStruct((ACC_ROWS, 16), jnp.float32)] * 2,
        mesh=mesh,
        compiler_params=pltpu.CompilerParams(use_tc_tiling_on_sc=False),
        scratch_types=[
            pltpu.VMEM_SHARED((ACC_ROWS, 16), jnp.float32),
            pltpu.VMEM((SLAB, CHUNK), jnp.int32),
            pltpu.VMEM((CHUNK, 16), jnp.float32),
        ],
    )


def _enc_body(x_ref, w_ref, b_ref, o0_ref, o1_ref):
    h = jnp.dot(x_ref[...], w_ref[...], preferred_element_type=jnp.float32)
    h = jnp.maximum(h + b_ref[...], 0.0)
    o0_ref[...] = h[:, :HALF]
    o1_ref[...] = h[:, HALF:]


def _encode(x, w, b):
    d = x.shape[1]
    return pl.pallas_call(
        _enc_body,
        grid=(GRID,),
        in_specs=[
            pl.BlockSpec((ROWBLK, d), lambda i: (i, 0)),
            pl.BlockSpec((d, HID), lambda i: (0, 0)),
            pl.BlockSpec((1, HID), lambda i: (0, 0)),
        ],
        out_specs=[pl.BlockSpec((ROWBLK, HALF), lambda i: (i, 0))] * 2,
        out_shape=[jax.ShapeDtypeStruct((N, HALF), jnp.float32)] * 2,
    )(x, w, b.reshape(1, HID))


def _dense_body(s0_ref, s1_ref, cnt_ref, x0_ref, x1_ref, wl_ref, bl_ref,
                wr_ref, y0_ref, y1_ref):
    cnt = jnp.maximum(cnt_ref[...][:, :1], 1.0)
    m = jnp.concatenate([s0_ref[...], s1_ref[...]], axis=1) / cnt
    x = jnp.concatenate([x0_ref[...], x1_ref[...]], axis=1)
    h = jnp.dot(m, wl_ref[...], preferred_element_type=jnp.float32)
    h = h + bl_ref[...]
    h = h + jnp.dot(x, wr_ref[...], preferred_element_type=jnp.float32)
    h = jnp.maximum(h, 0.0) + x
    y0_ref[...] = h[:, :HALF]
    y1_ref[...] = h[:, HALF:]


def _dense(s0, s1, cnt, x0, x1, wl, bl, wr):
    return pl.pallas_call(
        _dense_body,
        grid=(GRID,),
        in_specs=[
            pl.BlockSpec((ROWBLK, HALF), lambda i: (i, 0)),
            pl.BlockSpec((ROWBLK, HALF), lambda i: (i, 0)),
            pl.BlockSpec((ROWBLK, 16), lambda i: (i, 0)),
            pl.BlockSpec((ROWBLK, HALF), lambda i: (i, 0)),
            pl.BlockSpec((ROWBLK, HALF), lambda i: (i, 0)),
            pl.BlockSpec((HID, HID), lambda i: (0, 0)),
            pl.BlockSpec((1, HID), lambda i: (0, 0)),
            pl.BlockSpec((HID, HID), lambda i: (0, 0)),
        ],
        out_specs=[pl.BlockSpec((ROWBLK, HALF), lambda i: (i, 0))] * 2,
        out_shape=[jax.ShapeDtypeStruct((N, HALF), jnp.float32)] * 2,
    )(s0, s1, cnt, x0, x1, wl, bl.reshape(1, HID), wr)


def _dense_head_body(s0_ref, s1_ref, cnt_ref, x0_ref, x1_ref, wl_ref, bl_ref,
                     wr_ref, w1_ref, b1_ref, w2_ref, b2_ref, o_ref):
    cnt = jnp.maximum(cnt_ref[...][:, :1], 1.0)
    m = jnp.concatenate([s0_ref[...], s1_ref[...]], axis=1) / cnt
    x = jnp.concatenate([x0_ref[...], x1_ref[...]], axis=1)
    h = jnp.dot(m, wl_ref[...], preferred_element_type=jnp.float32)
    h = h + bl_ref[...]
    h = h + jnp.dot(x, wr_ref[...], preferred_element_type=jnp.float32)
    h = jnp.maximum(h, 0.0) + x
    hh = jnp.dot(h, w1_ref[...], preferred_element_type=jnp.float32)
    hh = jnp.maximum(hh + b1_ref[...], 0.0)
    o = jnp.dot(hh, w2_ref[...], preferred_element_type=jnp.float32)
    o_ref[...] = o + b2_ref[...]


def _dense_head(s0, s1, cnt, x0, x1, wl, bl, wr, w1, b1, w2, b2):
    return pl.pallas_call(
        _dense_head_body,
        grid=(GRID,),
        in_specs=[
            pl.BlockSpec((ROWBLK, HALF), lambda i: (i, 0)),
            pl.BlockSpec((ROWBLK, HALF), lambda i: (i, 0)),
            pl.BlockSpec((ROWBLK, 16), lambda i: (i, 0)),
            pl.BlockSpec((ROWBLK, HALF), lambda i: (i, 0)),
            pl.BlockSpec((ROWBLK, HALF), lambda i: (i, 0)),
            pl.BlockSpec((HID, HID), lambda i: (0, 0)),
            pl.BlockSpec((1, HID), lambda i: (0, 0)),
            pl.BlockSpec((HID, HID), lambda i: (0, 0)),
            pl.BlockSpec((HID, HID // 2), lambda i: (0, 0)),
            pl.BlockSpec((1, HID // 2), lambda i: (0, 0)),
            pl.BlockSpec((HID // 2, 1), lambda i: (0, 0)),
            pl.BlockSpec((1, 1), lambda i: (0, 0)),
        ],
        out_specs=pl.BlockSpec((ROWBLK, 1), lambda i: (i, 0)),
        out_shape=jax.ShapeDtypeStruct((N, 1), jnp.float32),
    )(s0, s1, cnt, x0, x1, wl, bl.reshape(1, HID), wr,
      w1, b1.reshape(1, HID // 2), w2, b2.reshape(1, 1))


def _head_body(x0_ref, x1_ref, w1_ref, b1_ref, w2_ref, b2_ref, o_ref):
    x = jnp.concatenate([x0_ref[...], x1_ref[...]], axis=1)
    h = jnp.dot(x, w1_ref[...], preferred_element_type=jnp.float32)
    h = jnp.maximum(h + b1_ref[...], 0.0)
    o = jnp.dot(h, w2_ref[...], preferred_element_type=jnp.float32)
    o_ref[...] = o + b2_ref[...]


def _head(x0, x1, w1, b1, w2, b2):
    return pl.pallas_call(
        _head_body,
        grid=(GRID,),
        in_specs=[
            pl.BlockSpec((ROWBLK, HALF), lambda i: (i, 0)),
            pl.BlockSpec((ROWBLK, HALF), lambda i: (i, 0)),
            pl.BlockSpec((HID, HID // 2), lambda i: (0, 0)),
            pl.BlockSpec((1, HID // 2), lambda i: (0, 0)),
            pl.BlockSpec((HID // 2, 1), lambda i: (0, 0)),
            pl.BlockSpec((1, 1), lambda i: (0, 0)),
        ],
        out_specs=pl.BlockSpec((ROWBLK, 1), lambda i: (i, 0)),
        out_shape=jax.ShapeDtypeStruct((N, 1), jnp.float32),
    )(x0, x1, w1, b1.reshape(1, HID // 2), w2, b2.reshape(1, 1))


def _pad_edges(e):
    pad = jnp.broadcast_to(
        jnp.array([[0], [SINK]], dtype=jnp.int32), (2, E_PAD - E))
    ep = jnp.concatenate([e, pad], axis=1)
    return (ep[0].reshape(NBLK, SLAB, CHUNK), ep[1].reshape(NBLK, SLAB, CHUNK))


@jax.jit
def kernel(x_compound, x_target, edge_ct, edge_tc, W_comp, b_comp, W_tgt,
           b_tgt, Wl_ct, bl_ct, Wr_ct, Wl_tc, bl_tc, Wr_tc, W_out1, b_out1,
           W_out2, b_out2):
    sct, dct = _pad_edges(edge_ct)
    stc, dtc = _pad_edges(edge_tc)
    zeros32 = jnp.zeros((CHUNK, HALF), jnp.float32)
    zeros16 = jnp.zeros((CHUNK, 16), jnp.float32)
    ones16 = jnp.ones((CHUNK, 16), jnp.float32)

    xc0, xc1 = _encode(x_compound, W_comp, b_comp)
    xt0, xt1 = _encode(x_target, W_tgt, b_tgt)

    cnt_t, cnt_c = _make_sc_cnt()(dct, dtc, ones16, zeros16)

    sc_agg = _make_sc_agg()
    L = Wl_ct.shape[0]
    for i in range(L - 1):
        st0, st1, sc0, sc1 = sc_agg(xc0, xc1, xt0, xt1, sct, dct, stc, dtc,
                                    zeros32)
        xt0, xt1 = _dense(st0, st1, cnt_t, xt0, xt1, Wl_ct[i], bl_ct[i],
                          Wr_ct[i])
        xc0, xc1 = _dense(sc0, sc1, cnt_c, xc0, xc1, Wl_tc[i], bl_tc[i],
                          Wr_tc[i])

    sc0, sc1 = _make_sc_agg1()(xt0, xt1, stc, dtc, zeros32)
    return _dense_head(sc0, sc1, cnt_c, xc0, xc1, Wl_tc[L - 1],
                       bl_tc[L - 1], Wr_tc[L - 1], W_out1, b_out1,
                       W_out2, b_out2)
